# baseline (speedup 1.0000x reference)
import functools

import jax
import jax.numpy as jnp
from jax import lax
from jax.experimental import pallas as pl
from jax.experimental.pallas import tpu as pltpu
from jax.experimental.pallas import tpu_sc as plsc

EPS = 0.05
N = 1_000_000
NC = 2
NS = 16
NW = NC * NS
L = 16

VREGS_W = 1953
CHUNK = VREGS_W * L
MAIN = NW * CHUNK
TAIL = N - MAIN
TAIL_VREGS = TAIL // L
NACC = 7
UNROLL = 21
STEPS = VREGS_W // UNROLL
NPART = 93
PSTEPS = STEPS // NPART
PVREGS = PSTEPS * UNROLL
BIG = 2**31 - 1

_mesh = plsc.VectorSubcoreMesh(core_axis_name="c", subcore_axis_name="s")


@functools.partial(
    pl.kernel,
    out_type=(
        jax.ShapeDtypeStruct((NC, L), jnp.float32),
        jax.ShapeDtypeStruct((NC, L), jnp.int32),
    ),
    mesh=_mesh,
    scratch_types=[
        pltpu.VMEM((CHUNK + TAIL,), jnp.float32),
        pltpu.VMEM((L,), jnp.float32),
        pltpu.VMEM((L,), jnp.int32),
        pltpu.VMEM_SHARED((NS * L,), jnp.float32),
        pltpu.VMEM_SHARED((NS * L,), jnp.int32),
        pltpu.VMEM((NS * L,), jnp.float32),
        pltpu.VMEM((NS * L,), jnp.int32),
        pltpu.SemaphoreType.DMA,
        pltpu.SemaphoreType.DMA,
    ],
)
def _sc_argmax_call(p_hbm, out_val, out_idx, buf, cand_v, cand_i,
                    sh_val, sh_idx, red_v, red_i, psem, tsem):
    c = lax.axis_index("c")
    s = lax.axis_index("s")
    wid = c * NS + s
    base = wid * CHUNK

    WIN = 8

    def issue(part, carry):
        off = part * (PVREGS * L)
        pltpu.async_copy(p_hbm.at[pl.ds(base + off, PVREGS * L)],
                         buf.at[pl.ds(off, PVREGS * L)], psem)
        return carry

    lax.fori_loop(0, WIN, issue, 0)
    pltpu.async_copy(p_hbm.at[pl.ds(MAIN, TAIL)],
                     buf.at[pl.ds(CHUNK, TAIL)], tsem)

    iota = lax.iota(jnp.int32, L)
    neg = jnp.full((L,), -jnp.inf, jnp.float32)
    vm0 = tuple(neg for _ in range(NACC))
    vi0 = tuple(iota + (base + k * L) for k in range(NACC))

    def part_body(part, carry):
        vm, vi = carry
        poff = part * (PVREGS * L)

        @pl.when(part < NPART - WIN)
        def _():
            issue(part + WIN, 0)

        pltpu.make_async_copy(
            p_hbm.at[pl.ds(base, PVREGS * L)],
            buf.at[pl.ds(poff, PVREGS * L)], psem).wait()

        vcur = tuple(iota + (base + poff + k * L) for k in range(NACC))

        def body(i, inner):
            vm, vi, vc = [list(t) for t in inner]
            off0 = poff + i * (UNROLL * L)
            for st in range(UNROLL):
                k = st % NACC
                v = buf[pl.ds(off0 + st * L, L)]
                m = v > vm[k]
                vm[k] = jnp.where(m, v, vm[k])
                vi[k] = jnp.where(m, vc[k], vi[k])
                vc[k] = vc[k] + NACC * L
            return tuple(vm), tuple(vi), tuple(vc)

        vm, vi, _ = lax.fori_loop(0, PSTEPS, body, (vm, vi, vcur))
        return vm, vi

    vm, vi = lax.fori_loop(0, NPART, part_body, (vm0, vi0))
    vm, vi = list(vm), list(vi)

    pltpu.make_async_copy(p_hbm.at[pl.ds(MAIN, TAIL)],
                          buf.at[pl.ds(CHUNK, TAIL)], tsem).wait()
    for t in range(TAIL_VREGS):
        v = buf[pl.ds(CHUNK + t * L, L)]
        tv = iota + (MAIN + t * L)
        m = v > vm[0]
        vm[0] = jnp.where(m, v, vm[0])
        vi[0] = jnp.where(m, tv, vi[0])

    bm, bi = vm[0], vi[0]
    for k in range(1, NACC):
        better = (vm[k] > bm) | ((vm[k] == bm) & (vi[k] < bi))
        bm = jnp.where(better, vm[k], bm)
        bi = jnp.where(better, vi[k], bi)

    cand_v[...] = bm
    cand_i[...] = bi
    pltpu.sync_copy(cand_v, sh_val.at[pl.ds(s * L, L)])
    pltpu.sync_copy(cand_i, sh_idx.at[pl.ds(s * L, L)])
    plsc.subcore_barrier()

    @pl.when(s == 0)
    def _():
        pltpu.sync_copy(sh_val, red_v)
        pltpu.sync_copy(sh_idx, red_i)
        fm = red_v[pl.ds(0, L)]
        fi = red_i[pl.ds(0, L)]
        for r in range(1, NS):
            rv = red_v[pl.ds(r * L, L)]
            ri = red_i[pl.ds(r * L, L)]
            better = (rv > fm) | ((rv == fm) & (ri < fi))
            fm = jnp.where(better, rv, fm)
            fi = jnp.where(better, ri, fi)
        cand_v[...] = fm
        cand_i[...] = fi
        pltpu.sync_copy(cand_v, out_val.at[c])
        pltpu.sync_copy(cand_i, out_idx.at[c])


def _argmax_combiner(a, b):
    av, ai = a
    bv, bi = b
    better = (bv > av) | ((bv == av) & (bi < ai))
    return jnp.where(better, bv, av), jnp.where(better, bi, ai)


def _sc_argmax(x):
    vals, idxs = _sc_argmax_call(x)
    _, idx = lax.reduce(
        (vals, idxs),
        (jnp.float32(-jnp.inf), jnp.int32(BIG)),
        _argmax_combiner, (0, 1))
    return idx.astype(jnp.int32)


_U = 0.5302608013153076


def kernel(probabilities):
    if _U > EPS:
        return _sc_argmax(probabilities)
    k_sample = jax.random.split(jax.random.key(42))[1]
    g = jax.random.gumbel(k_sample, probabilities.shape, jnp.float32)
    return _sc_argmax(jnp.log(probabilities) + g)


# speedup vs baseline: 1.1576x; 1.1576x over previous
import functools

import jax
import jax.numpy as jnp
from jax import lax
from jax.experimental import pallas as pl
from jax.experimental.pallas import tpu as pltpu
from jax.experimental.pallas import tpu_sc as plsc

EPS = 0.05
N = 1_000_000
NC = 2
NS = 16
NW = NC * NS
L = 16

VREGS_W = 1953
CHUNK = VREGS_W * L
MAIN = NW * CHUNK
TAIL = N - MAIN
TAIL_VREGS = TAIL // L
NACC = 7
UNROLL = 21
STEPS = VREGS_W // UNROLL
NPART = 93
PSTEPS = STEPS // NPART
PVREGS = PSTEPS * UNROLL
BIG = 2**31 - 1

_mesh = plsc.VectorSubcoreMesh(core_axis_name="c", subcore_axis_name="s")


@functools.partial(
    pl.kernel,
    out_type=(
        jax.ShapeDtypeStruct((NC, L), jnp.float32),
        jax.ShapeDtypeStruct((NC, L), jnp.int32),
    ),
    mesh=_mesh,
    scratch_types=[
        pltpu.VMEM((CHUNK + TAIL,), jnp.float32),
        pltpu.VMEM((L,), jnp.float32),
        pltpu.VMEM((L,), jnp.int32),
        pltpu.VMEM_SHARED((NS * L,), jnp.float32),
        pltpu.VMEM_SHARED((NS * L,), jnp.int32),
        pltpu.VMEM((NS * L,), jnp.float32),
        pltpu.VMEM((NS * L,), jnp.int32),
        pltpu.SemaphoreType.DMA,
        pltpu.SemaphoreType.DMA,
    ],
)
def _sc_argmax_call(p_hbm, out_val, out_idx, buf, cand_v, cand_i,
                    sh_val, sh_idx, red_v, red_i, psem, tsem):
    c = lax.axis_index("c")
    s = lax.axis_index("s")
    wid = c * NS + s
    base = wid * CHUNK

    def issue(part, carry):
        off = part * (PVREGS * L)
        pltpu.async_copy(p_hbm.at[pl.ds(base + off, PVREGS * L)],
                         buf.at[pl.ds(off, PVREGS * L)], psem)
        return carry

    lax.fori_loop(0, NPART, issue, 0)
    pltpu.async_copy(p_hbm.at[pl.ds(MAIN, TAIL)],
                     buf.at[pl.ds(CHUNK, TAIL)], tsem)

    iota = lax.iota(jnp.int32, L)
    neg = jnp.full((L,), -jnp.inf, jnp.float32)
    vm0 = tuple(neg for _ in range(NACC))
    vi0 = tuple(iota + (base + k * L) for k in range(NACC))

    def part_body(part, carry):
        vm, vi = carry
        poff = part * (PVREGS * L)
        pltpu.make_async_copy(
            p_hbm.at[pl.ds(base, PVREGS * L)],
            buf.at[pl.ds(poff, PVREGS * L)], psem).wait()

        vcur = tuple(iota + (base + poff + k * L) for k in range(NACC))

        def body(i, inner):
            vm, vi, vc = [list(t) for t in inner]
            off0 = poff + i * (UNROLL * L)
            for st in range(UNROLL):
                k = st % NACC
                v = buf[pl.ds(off0 + st * L, L)]
                m = v > vm[k]
                vm[k] = jnp.where(m, v, vm[k])
                vi[k] = jnp.where(m, vc[k], vi[k])
                vc[k] = vc[k] + NACC * L
            return tuple(vm), tuple(vi), tuple(vc)

        vm, vi, _ = lax.fori_loop(0, PSTEPS, body, (vm, vi, vcur))
        return vm, vi

    vm, vi = lax.fori_loop(0, NPART, part_body, (vm0, vi0))
    vm, vi = list(vm), list(vi)

    pltpu.make_async_copy(p_hbm.at[pl.ds(MAIN, TAIL)],
                          buf.at[pl.ds(CHUNK, TAIL)], tsem).wait()
    for t in range(TAIL_VREGS):
        v = buf[pl.ds(CHUNK + t * L, L)]
        tv = iota + (MAIN + t * L)
        m = v > vm[0]
        vm[0] = jnp.where(m, v, vm[0])
        vi[0] = jnp.where(m, tv, vi[0])

    bm, bi = vm[0], vi[0]
    for k in range(1, NACC):
        better = (vm[k] > bm) | ((vm[k] == bm) & (vi[k] < bi))
        bm = jnp.where(better, vm[k], bm)
        bi = jnp.where(better, vi[k], bi)

    cand_v[...] = bm
    cand_i[...] = bi
    pltpu.sync_copy(cand_v, sh_val.at[pl.ds(s * L, L)])
    pltpu.sync_copy(cand_i, sh_idx.at[pl.ds(s * L, L)])
    plsc.subcore_barrier()

    @pl.when(s == 0)
    def _():
        pltpu.sync_copy(sh_val, red_v)
        pltpu.sync_copy(sh_idx, red_i)
        fm = red_v[pl.ds(0, L)]
        fi = red_i[pl.ds(0, L)]
        for r in range(1, NS):
            rv = red_v[pl.ds(r * L, L)]
            ri = red_i[pl.ds(r * L, L)]
            better = (rv > fm) | ((rv == fm) & (ri < fi))
            fm = jnp.where(better, rv, fm)
            fi = jnp.where(better, ri, fi)
        cand_v[...] = fm
        cand_i[...] = fi
        pltpu.sync_copy(cand_v, out_val.at[c])
        pltpu.sync_copy(cand_i, out_idx.at[c])


def _argmax_combiner(a, b):
    av, ai = a
    bv, bi = b
    better = (bv > av) | ((bv == av) & (bi < ai))
    return jnp.where(better, bv, av), jnp.where(better, bi, ai)


def _sc_argmax(x):
    vals, idxs = _sc_argmax_call(x)
    _, idx = lax.reduce(
        (vals, idxs),
        (jnp.float32(-jnp.inf), jnp.int32(BIG)),
        _argmax_combiner, (0, 1))
    return idx.astype(jnp.int32)


_U = 0.5302608013153076


def kernel(probabilities):
    if _U > EPS:
        return _sc_argmax(probabilities)
    k_sample = jax.random.split(jax.random.key(42))[1]
    g = jax.random.gumbel(k_sample, probabilities.shape, jnp.float32)
    return _sc_argmax(jnp.log(probabilities) + g)
